# Initial kernel scaffold; baseline (speedup 1.0000x reference)
import jax
import jax.numpy as jnp
from jax.experimental import pallas as pl


def kernel(in_feat, edge_index, graph_ids, gcn_emb, gin_emb, gcn_W, gcn_b, gin_W1, gin_b1, gin_W2, gin_b2, gin_eps, ffnn_W0, ffnn_b0, ffnn_W1, ffnn_b1, fc_W, fc_b):
    raise NotImplementedError("write your pallas kernel here")


# speedup vs baseline: 2.7290x; 2.7290x over previous
import functools

import jax
import jax.numpy as jnp
from jax import lax
from jax.experimental import pallas as pl
from jax.experimental.pallas import tpu as pltpu
from jax.experimental.pallas import tpu_sc as plsc

N = 10000
E = 320000
D = 128
G = 64
NPAD = 10240
CPT = 160
NCHUNK = 16 * CPT
ROWS_PT = NPAD // 16
EMB_CPW = 3
NEMB = 32 * EMB_CPW * 128
NBLK = NPAD // 1024

_MESH = dict(core_axis_name="c", subcore_axis_name="s", num_cores=2,
             num_subcores=16)


def _sc_prep_body(e2d, feat3d, gcn_emb, gin_emb,
                  emb_gcn, emb_gin, degs,
                  idx_e, acc1d, feat_idx, gbuf, zbuf, ones_row, sem):
    c = lax.axis_index("c")
    s = lax.axis_index("s")
    w = s * 2 + c
    zero = jnp.zeros((16,), jnp.float32)
    ones = jnp.ones((16,), jnp.float32)

    pltpu.sync_copy(feat3d.at[w], feat_idx)
    for k in range(EMB_CPW):
        row0 = (w * EMB_CPW + k) * 128
        pltpu.async_copy(gcn_emb.at[feat_idx.at[k]], gbuf, sem).wait()
        pltpu.sync_copy(gbuf, emb_gcn.at[pl.ds(row0, 128)])
        pltpu.async_copy(gin_emb.at[feat_idx.at[k]], gbuf, sem).wait()
        pltpu.sync_copy(gbuf, emb_gin.at[pl.ds(row0, 128)])

    pltpu.sync_copy(e2d.at[c].at[s], idx_e)

    @pl.loop(0, 40)
    def _zero_zbuf(r):
        zbuf[pl.ds(r * 16, 16)] = zero
    for k in range(8):
        ones_row[pl.ds(k * 16, 16)] = ones
    pltpu.sync_copy(zbuf, acc1d.at[pl.ds(640 * s, 640)])
    plsc.subcore_barrier()

    @pl.loop(0, CPT)
    def _count(j):
        pltpu.async_copy(ones_row, acc1d.at[idx_e.at[j]], sem, add=True)

    @pl.loop(0, CPT)
    def _drain(j):
        pltpu.make_async_copy(ones_row, acc1d.at[idx_e.at[j]], sem).wait()

    plsc.subcore_barrier()
    pltpu.sync_copy(acc1d.at[pl.ds(640 * s, 640)], zbuf)
    pltpu.sync_copy(zbuf, degs.at[c].at[pl.ds(640 * s, 640)])


_sc_prep = functools.partial(
    pl.kernel,
    out_type=(
        jax.ShapeDtypeStruct((NEMB, 128), jnp.float32),
        jax.ShapeDtypeStruct((NEMB, 128), jnp.float32),
        jax.ShapeDtypeStruct((2, NPAD), jnp.float32),
    ),
    mesh=plsc.VectorSubcoreMesh(**_MESH),
    scratch_types=[
        pltpu.VMEM((CPT, 128), jnp.int32),
        pltpu.VMEM_SHARED((NPAD,), jnp.float32),
        pltpu.VMEM((EMB_CPW, 128), jnp.int32),
        pltpu.VMEM((128, 128), jnp.float32),
        pltpu.VMEM((640,), jnp.float32),
        pltpu.VMEM((128,), jnp.float32),
        pltpu.SemaphoreType.DMA,
    ],
)(_sc_prep_body)


SEG = 16


def _sc_pass_body(e2d, A, zrows, M,
                  src_idx, dst_idx, b0, b1, acc, gsem, ssem):
    c = lax.axis_index("c")
    s = lax.axis_index("s")
    pltpu.sync_copy(zrows, acc.at[pl.ds(s * ROWS_PT, ROWS_PT)])
    plsc.subcore_barrier()

    tab = A.at[c]
    my_src = e2d.at[0].at[s]
    my_dst = e2d.at[1].at[s]

    def gather(j, buf):
        return pltpu.async_copy(tab.at[src_idx.at[j]], buf, gsem)

    def scat(j, buf):
        return pltpu.async_copy(buf, acc.at[dst_idx.at[j]], ssem, add=True)

    @pl.loop(0, CPT // SEG)
    def _segment(g):
        pltpu.sync_copy(my_src.at[pl.ds(g * SEG, SEG)], src_idx)
        pltpu.sync_copy(my_dst.at[pl.ds(g * SEG, SEG)], dst_idx)
        gather(0, b0)

        @pl.loop(0, SEG, step=2)
        def _pipeline(j):
            pltpu.make_async_copy(tab.at[src_idx.at[j]], b0, gsem).wait()
            sd0 = scat(j, b0)
            gd1 = gather(j + 1, b1)
            gd1.wait()
            sd1 = scat(j + 1, b1)
            sd0.wait()
            sd1.wait()

            @pl.when(j + 2 < SEG)
            def _():
                gather(j + 2, b0)

    plsc.subcore_barrier()
    pltpu.sync_copy(acc.at[pl.ds(s * ROWS_PT, ROWS_PT)],
                    M.at[c].at[pl.ds(s * ROWS_PT, ROWS_PT)])


_sc_pass = functools.partial(
    pl.kernel,
    out_type=jax.ShapeDtypeStruct((2, NPAD, 128), jnp.float32),
    mesh=plsc.VectorSubcoreMesh(**_MESH),
    scratch_types=[
        pltpu.VMEM((SEG, 128), jnp.int32),
        pltpu.VMEM((SEG, 128), jnp.int32),
        pltpu.VMEM((128, 128), jnp.float32),
        pltpu.VMEM((128, 128), jnp.float32),
        pltpu.VMEM_SHARED((NPAD, 128), jnp.float32),
        pltpu.SemaphoreType.DMA,
        pltpu.SemaphoreType.DMA,
    ],
)(_sc_pass_body)


def _norm(d):
    return jnp.where(d > 0, lax.rsqrt(d), 0.0)


def _mm(a, b):
    return jnp.dot(a, b, preferred_element_type=jnp.float32)


def _tc0_body(gcnh, ginh, dego, W0, A0):
    ns = _norm(dego[...])
    A0[0] = _mm(gcnh[...] * ns, W0[...])
    A0[1] = ginh[...]


def _gin_update(M1, ginh_prev, eps, W1, b1, W2, b2):
    h = (1.0 + eps[0, 0]) * ginh_prev + M1
    return _mm(jnp.maximum(_mm(h, W1[...]) + b1[...], 0.0), W2[...]) + b2[...]


def _tc_layer_body(Mr, Ar, dego, degi, gcnb, eps, W1, b1, W2, b2, Wn, Ao):
    gcn_h = jnp.maximum(Mr[0] * _norm(degi[...]) + gcnb[...], 0.0)
    Ao[0] = _mm(gcn_h * _norm(dego[...]), Wn[...])
    Ao[1] = _gin_update(Mr[1], Ar[1], eps, W1, b1, W2, b2)


def _tc_final_body(Mr, Ar, degi, gcnb, eps, W1, b1, W2, b2, gids,
                   fW0, fb0, fW1, fb1, fcW, fcb, out, accg, accn, cnt):
    j = pl.program_id(0)
    gcn_h = jnp.maximum(Mr[0] * _norm(degi[...]) + gcnb[...], 0.0)
    gin_h = _gin_update(Mr[1], Ar[1], eps, W1, b1, W2, b2)
    gid = gids[0, 0, :]
    iota_g = lax.broadcasted_iota(jnp.int32, (G, 1024), 0)
    mask = (iota_g == gid[None, :]).astype(jnp.float32)
    pg = _mm(mask, gcn_h)
    pn = _mm(mask, gin_h)
    cs = jnp.sum(mask, axis=1, keepdims=True)

    @pl.when(j == 0)
    def _():
        accg[...] = pg
        accn[...] = pn
        cnt[...] = cs

    @pl.when(j > 0)
    def _():
        accg[...] += pg
        accn[...] += pn
        cnt[...] += cs

    @pl.when(j == NBLK - 1)
    def _():
        c_ = jnp.maximum(cnt[...], 1.0)
        pooled = jnp.concatenate([accg[...] / c_, accn[...] / c_], axis=1)
        h = jnp.maximum(_mm(pooled, fW0[...]) + fb0[...], 0.0)
        h = jnp.maximum(_mm(h, fW1[...]) + fb1[...], 0.0)
        out[...] = jax.nn.sigmoid(_mm(h, fcW[...]) + fcb[...])


def _row_spec(last=128):
    return pl.BlockSpec((1024, last), lambda j: (j, 0))


def _full(shape):
    return pl.BlockSpec(shape, lambda j: tuple(0 for _ in shape))


def _plane_spec():
    return pl.BlockSpec((2, 1024, 128), lambda j: (0, j, 0))


def _tc0(gcnh, ginh, dego, W0):
    return pl.pallas_call(
        _tc0_body,
        grid=(NBLK,),
        in_specs=[_row_spec(), _row_spec(), _row_spec(1), _full((128, 128))],
        out_specs=_plane_spec(),
        out_shape=jax.ShapeDtypeStruct((2, NPAD, 128), jnp.float32),
    )(gcnh, ginh, dego, W0)


def _tc_layer(M, A, dego, degi, gcnb, eps, W1, b1, W2, b2, Wn):
    return pl.pallas_call(
        _tc_layer_body,
        grid=(NBLK,),
        in_specs=[_plane_spec(), _plane_spec(), _row_spec(1), _row_spec(1),
                  _full((1, 128)), _full((1, 1)), _full((128, 128)),
                  _full((1, 128)), _full((128, 128)), _full((1, 128)),
                  _full((128, 128))],
        out_specs=_plane_spec(),
        out_shape=jax.ShapeDtypeStruct((2, NPAD, 128), jnp.float32),
    )(M, A, dego, degi, gcnb, eps, W1, b1, W2, b2, Wn)


def _tc_final(M, A, degi, gcnb, eps, W1, b1, W2, b2, gids3d,
              fW0, fb0, fW1, fb1, fcW, fcb):
    return pl.pallas_call(
        _tc_final_body,
        grid=(NBLK,),
        in_specs=[_plane_spec(), _plane_spec(), _row_spec(1),
                  _full((1, 128)), _full((1, 1)), _full((128, 128)),
                  _full((1, 128)), _full((128, 128)), _full((1, 128)),
                  pl.BlockSpec((1, 1, 1024), lambda j: (j, 0, 0)),
                  _full((256, 128)), _full((1, 128)), _full((128, 128)),
                  _full((1, 128)), _full((128, 1)), _full((1, 1))],
        out_specs=_full((G, 1)),
        out_shape=jax.ShapeDtypeStruct((G, 1), jnp.float32),
        scratch_shapes=[pltpu.VMEM((G, 128), jnp.float32),
                        pltpu.VMEM((G, 128), jnp.float32),
                        pltpu.VMEM((G, 1), jnp.float32)],
    )(M, A, degi, gcnb, eps, W1, b1, W2, b2, gids3d,
      fW0, fb0, fW1, fb1, fcW, fcb)


def kernel(in_feat, edge_index, graph_ids, gcn_emb, gin_emb, gcn_W, gcn_b,
           gin_W1, gin_b1, gin_W2, gin_b2, gin_eps,
           ffnn_W0, ffnn_b0, ffnn_W1, ffnn_b1, fc_W, fc_b):
    f32 = jnp.float32
    ei = edge_index.astype(jnp.int32)
    e2d = jnp.pad(ei, ((0, 0), (0, NCHUNK * 128 - E)),
                  constant_values=N).reshape(2, 16, CPT, 128)
    feat3d = jnp.pad(in_feat[:, 0].astype(jnp.int32),
                     (0, NEMB - N)).reshape(32, EMB_CPW, 128)
    gids3d = jnp.pad(graph_ids.astype(jnp.int32), (0, NPAD - N),
                     constant_values=G).reshape(NBLK, 1, 1024)
    zrows = jnp.zeros((ROWS_PT, 128), f32)

    emb_gcn, emb_gin, degs = _sc_prep(e2d, feat3d, gcn_emb, gin_emb)
    dego = degs[0].reshape(NPAD, 1)
    degi = degs[1].reshape(NPAD, 1)

    b1 = gin_b1.reshape(1, 128)
    b2 = gin_b2.reshape(1, 128)

    A = _tc0(emb_gcn, emb_gin, dego, gcn_W[0])
    out = None
    for i in range(3):
        M = _sc_pass(e2d, A, zrows)
        eps = gin_eps[i].reshape(1, 1)
        gcnb = gcn_b[i].reshape(1, 128)
        if i < 2:
            A = _tc_layer(M, A, dego, degi, gcnb, eps,
                          gin_W1, b1, gin_W2, b2, gcn_W[i + 1])
        else:
            out = _tc_final(M, A, degi, gcnb, eps, gin_W1, b1, gin_W2, b2,
                            gids3d, ffnn_W0, ffnn_b0.reshape(1, 128),
                            ffnn_W1, ffnn_b1.reshape(1, 128),
                            fc_W, fc_b.reshape(1, 1))
    return out.reshape(G, 1, 1)


# speedup vs baseline: 2.9007x; 1.0629x over previous
import functools

import jax
import jax.numpy as jnp
from jax import lax
from jax.experimental import pallas as pl
from jax.experimental.pallas import tpu as pltpu
from jax.experimental.pallas import tpu_sc as plsc

N = 10000
E = 320000
D = 128
G = 64
NPAD = 10240
CPT = 160
NCHUNK = 16 * CPT
ROWS_PT = NPAD // 16
EMB_CPW = 3
NEMB = 32 * EMB_CPW * 128
NBLK = NPAD // 1024

_MESH = dict(core_axis_name="c", subcore_axis_name="s", num_cores=2,
             num_subcores=16)


def _sc_prep_body(e2d, feat3d, gcn_emb, gin_emb,
                  emb_gcn, emb_gin, degs,
                  idx_e, acc1d, feat_idx, gbuf, zbuf, ones_row, sem):
    c = lax.axis_index("c")
    s = lax.axis_index("s")
    w = s * 2 + c
    zero = jnp.zeros((16,), jnp.float32)
    ones = jnp.ones((16,), jnp.float32)

    pltpu.sync_copy(feat3d.at[w], feat_idx)
    for k in range(EMB_CPW):
        row0 = (w * EMB_CPW + k) * 128
        pltpu.async_copy(gcn_emb.at[feat_idx.at[k]], gbuf, sem).wait()
        pltpu.sync_copy(gbuf, emb_gcn.at[pl.ds(row0, 128)])
        pltpu.async_copy(gin_emb.at[feat_idx.at[k]], gbuf, sem).wait()
        pltpu.sync_copy(gbuf, emb_gin.at[pl.ds(row0, 128)])

    pltpu.sync_copy(e2d.at[c].at[s], idx_e)

    @pl.loop(0, 40)
    def _zero_zbuf(r):
        zbuf[pl.ds(r * 16, 16)] = zero
    for k in range(8):
        ones_row[pl.ds(k * 16, 16)] = ones
    pltpu.sync_copy(zbuf, acc1d.at[pl.ds(640 * s, 640)])
    plsc.subcore_barrier()

    @pl.loop(0, CPT)
    def _count(j):
        pltpu.async_copy(ones_row, acc1d.at[idx_e.at[j]], sem, add=True)

    @pl.loop(0, CPT)
    def _drain(j):
        pltpu.make_async_copy(ones_row, acc1d.at[idx_e.at[j]], sem).wait()

    plsc.subcore_barrier()
    pltpu.sync_copy(acc1d.at[pl.ds(640 * s, 640)], zbuf)
    pltpu.sync_copy(zbuf, degs.at[c].at[pl.ds(640 * s, 640)])


_sc_prep = functools.partial(
    pl.kernel,
    out_type=(
        jax.ShapeDtypeStruct((NEMB, 128), jnp.float32),
        jax.ShapeDtypeStruct((NEMB, 128), jnp.float32),
        jax.ShapeDtypeStruct((2, NPAD), jnp.float32),
    ),
    mesh=plsc.VectorSubcoreMesh(**_MESH),
    scratch_types=[
        pltpu.VMEM((CPT, 128), jnp.int32),
        pltpu.VMEM_SHARED((NPAD,), jnp.float32),
        pltpu.VMEM((EMB_CPW, 128), jnp.int32),
        pltpu.VMEM((128, 128), jnp.float32),
        pltpu.VMEM((640,), jnp.float32),
        pltpu.VMEM((128,), jnp.float32),
        pltpu.SemaphoreType.DMA,
    ],
)(_sc_prep_body)


CHUNK = 64
CPT2 = 320
SEG = 32
NSEG = CPT2 // SEG


def _sc_pass_body(e2d, A, zrows, M,
                  sidx, didx, b0, b1, b2, b3, acc,
                  g0, g1, g2, g3, s0, s1, s2, s3, isem):
    c = lax.axis_index("c")
    s = lax.axis_index("s")
    pltpu.sync_copy(zrows, acc.at[pl.ds(s * ROWS_PT, ROWS_PT)])
    plsc.subcore_barrier()

    tab = A.at[c]
    my_src = e2d.at[0].at[s]
    my_dst = e2d.at[1].at[s]
    bufs = (b0, b1, b2, b3)
    gsems = (g0, g1, g2, g3)
    ssems = (s0, s1, s2, s3)

    pltpu.sync_copy(my_src.at[pl.ds(0, SEG)], sidx.at[0])
    pltpu.sync_copy(my_dst.at[pl.ds(0, SEG)], didx.at[0])

    for g in range(NSEG):
        p = g & 1
        sp = sidx.at[p]
        dp = didx.at[p]
        if g >= 1:
            pltpu.make_async_copy(my_src.at[pl.ds(g * SEG, SEG)],
                                  sp, isem).wait()
            pltpu.make_async_copy(my_dst.at[pl.ds(g * SEG, SEG)],
                                  dp, isem).wait()
        if g + 1 < NSEG:
            pltpu.async_copy(my_src.at[pl.ds((g + 1) * SEG, SEG)],
                             sidx.at[1 - p], isem)
            pltpu.async_copy(my_dst.at[pl.ds((g + 1) * SEG, SEG)],
                             didx.at[1 - p], isem)

        def gath(row, k):
            pltpu.async_copy(tab.at[sp.at[row]], bufs[k], gsems[k])

        def wait_gath(row, k):
            pltpu.make_async_copy(tab.at[sp.at[row]], bufs[k],
                                  gsems[k]).wait()

        def scat(row, k):
            pltpu.async_copy(bufs[k], acc.at[dp.at[row]], ssems[k], add=True)

        def wait_scat(row, k):
            pltpu.make_async_copy(bufs[k], acc.at[dp.at[row]],
                                  ssems[k]).wait()

        gath(0, 0)
        gath(1, 1)
        gath(2, 2)

        for k in range(4):
            wait_gath(k, k)
            scat(k, k)
            if k >= 1:
                wait_scat(k - 1, (k + 3) % 4)
            gath(k + 3, (k + 3) % 4)

        @pl.loop(4, SEG - 4, step=4)
        def _quad(j):
            for k in range(4):
                jj = j + k
                wait_gath(jj, k)
                scat(jj, k)
                wait_scat(jj - 1, (k + 3) % 4)
                gath(jj + 3, (k + 3) % 4)

        for k in range(4):
            jj = SEG - 4 + k
            wait_gath(jj, k)
            scat(jj, k)
            if k == 0:
                wait_scat(jj - 1, 3)
                gath(jj + 3, 3)
        for k in range(4):
            wait_scat(SEG - 4 + k, k)

    plsc.subcore_barrier()
    pltpu.sync_copy(acc.at[pl.ds(s * ROWS_PT, ROWS_PT)],
                    M.at[c].at[pl.ds(s * ROWS_PT, ROWS_PT)])


_sc_pass = functools.partial(
    pl.kernel,
    out_type=jax.ShapeDtypeStruct((2, NPAD, 128), jnp.float32),
    mesh=plsc.VectorSubcoreMesh(**_MESH),
    scratch_types=[
        pltpu.VMEM((2, SEG, CHUNK), jnp.int32),
        pltpu.VMEM((2, SEG, CHUNK), jnp.int32),
        pltpu.VMEM((CHUNK, 128), jnp.float32),
        pltpu.VMEM((CHUNK, 128), jnp.float32),
        pltpu.VMEM((CHUNK, 128), jnp.float32),
        pltpu.VMEM((CHUNK, 128), jnp.float32),
        pltpu.VMEM_SHARED((NPAD, 128), jnp.float32),
        pltpu.SemaphoreType.DMA,
        pltpu.SemaphoreType.DMA,
        pltpu.SemaphoreType.DMA,
        pltpu.SemaphoreType.DMA,
        pltpu.SemaphoreType.DMA,
        pltpu.SemaphoreType.DMA,
        pltpu.SemaphoreType.DMA,
        pltpu.SemaphoreType.DMA,
        pltpu.SemaphoreType.DMA,
    ],
)(_sc_pass_body)


def _norm(d):
    return jnp.where(d > 0, lax.rsqrt(d), 0.0)


def _mm(a, b):
    return jnp.dot(a, b, preferred_element_type=jnp.float32)


def _tc0_body(gcnh, ginh, dego, W0, A0):
    ns = _norm(dego[...])
    A0[0] = _mm(gcnh[...] * ns, W0[...])
    A0[1] = ginh[...]


def _gin_update(M1, ginh_prev, eps, W1, b1, W2, b2):
    h = (1.0 + eps[0, 0]) * ginh_prev + M1
    return _mm(jnp.maximum(_mm(h, W1[...]) + b1[...], 0.0), W2[...]) + b2[...]


def _tc_layer_body(Mr, Ar, dego, degi, gcnb, eps, W1, b1, W2, b2, Wn, Ao):
    gcn_h = jnp.maximum(Mr[0] * _norm(degi[...]) + gcnb[...], 0.0)
    Ao[0] = _mm(gcn_h * _norm(dego[...]), Wn[...])
    Ao[1] = _gin_update(Mr[1], Ar[1], eps, W1, b1, W2, b2)


def _tc_final_body(Mr, Ar, degi, gcnb, eps, W1, b1, W2, b2, gids,
                   fW0, fb0, fW1, fb1, fcW, fcb, out, accg, accn, cnt):
    j = pl.program_id(0)
    gcn_h = jnp.maximum(Mr[0] * _norm(degi[...]) + gcnb[...], 0.0)
    gin_h = _gin_update(Mr[1], Ar[1], eps, W1, b1, W2, b2)
    gid = gids[0, 0, :]
    iota_g = lax.broadcasted_iota(jnp.int32, (G, 1024), 0)
    mask = (iota_g == gid[None, :]).astype(jnp.float32)
    pg = _mm(mask, gcn_h)
    pn = _mm(mask, gin_h)
    cs = jnp.sum(mask, axis=1, keepdims=True)

    @pl.when(j == 0)
    def _():
        accg[...] = pg
        accn[...] = pn
        cnt[...] = cs

    @pl.when(j > 0)
    def _():
        accg[...] += pg
        accn[...] += pn
        cnt[...] += cs

    @pl.when(j == NBLK - 1)
    def _():
        c_ = jnp.maximum(cnt[...], 1.0)
        pooled = jnp.concatenate([accg[...] / c_, accn[...] / c_], axis=1)
        h = jnp.maximum(_mm(pooled, fW0[...]) + fb0[...], 0.0)
        h = jnp.maximum(_mm(h, fW1[...]) + fb1[...], 0.0)
        out[...] = jax.nn.sigmoid(_mm(h, fcW[...]) + fcb[...])


def _row_spec(last=128):
    return pl.BlockSpec((1024, last), lambda j: (j, 0))


def _full(shape):
    return pl.BlockSpec(shape, lambda j: tuple(0 for _ in shape))


def _plane_spec():
    return pl.BlockSpec((2, 1024, 128), lambda j: (0, j, 0))


def _tc0(gcnh, ginh, dego, W0):
    return pl.pallas_call(
        _tc0_body,
        grid=(NBLK,),
        in_specs=[_row_spec(), _row_spec(), _row_spec(1), _full((128, 128))],
        out_specs=_plane_spec(),
        out_shape=jax.ShapeDtypeStruct((2, NPAD, 128), jnp.float32),
    )(gcnh, ginh, dego, W0)


def _tc_layer(M, A, dego, degi, gcnb, eps, W1, b1, W2, b2, Wn):
    return pl.pallas_call(
        _tc_layer_body,
        grid=(NBLK,),
        in_specs=[_plane_spec(), _plane_spec(), _row_spec(1), _row_spec(1),
                  _full((1, 128)), _full((1, 1)), _full((128, 128)),
                  _full((1, 128)), _full((128, 128)), _full((1, 128)),
                  _full((128, 128))],
        out_specs=_plane_spec(),
        out_shape=jax.ShapeDtypeStruct((2, NPAD, 128), jnp.float32),
    )(M, A, dego, degi, gcnb, eps, W1, b1, W2, b2, Wn)


def _tc_final(M, A, degi, gcnb, eps, W1, b1, W2, b2, gids3d,
              fW0, fb0, fW1, fb1, fcW, fcb):
    return pl.pallas_call(
        _tc_final_body,
        grid=(NBLK,),
        in_specs=[_plane_spec(), _plane_spec(), _row_spec(1),
                  _full((1, 128)), _full((1, 1)), _full((128, 128)),
                  _full((1, 128)), _full((128, 128)), _full((1, 128)),
                  pl.BlockSpec((1, 1, 1024), lambda j: (j, 0, 0)),
                  _full((256, 128)), _full((1, 128)), _full((128, 128)),
                  _full((1, 128)), _full((128, 1)), _full((1, 1))],
        out_specs=_full((G, 1)),
        out_shape=jax.ShapeDtypeStruct((G, 1), jnp.float32),
        scratch_shapes=[pltpu.VMEM((G, 128), jnp.float32),
                        pltpu.VMEM((G, 128), jnp.float32),
                        pltpu.VMEM((G, 1), jnp.float32)],
    )(M, A, degi, gcnb, eps, W1, b1, W2, b2, gids3d,
      fW0, fb0, fW1, fb1, fcW, fcb)


def kernel(in_feat, edge_index, graph_ids, gcn_emb, gin_emb, gcn_W, gcn_b,
           gin_W1, gin_b1, gin_W2, gin_b2, gin_eps,
           ffnn_W0, ffnn_b0, ffnn_W1, ffnn_b1, fc_W, fc_b):
    f32 = jnp.float32
    ei = edge_index.astype(jnp.int32)
    e_flat = jnp.pad(ei, ((0, 0), (0, NCHUNK * 128 - E)), constant_values=N)
    e2d = e_flat.reshape(2, 16, CPT, 128)
    e2d_p = e_flat.reshape(2, 16, CPT2, CHUNK)
    feat3d = jnp.pad(in_feat[:, 0].astype(jnp.int32),
                     (0, NEMB - N)).reshape(32, EMB_CPW, 128)
    gids3d = jnp.pad(graph_ids.astype(jnp.int32), (0, NPAD - N),
                     constant_values=G).reshape(NBLK, 1, 1024)
    zrows = jnp.zeros((ROWS_PT, 128), f32)

    emb_gcn, emb_gin, degs = _sc_prep(e2d, feat3d, gcn_emb, gin_emb)
    dego = degs[0].reshape(NPAD, 1)
    degi = degs[1].reshape(NPAD, 1)

    b1 = gin_b1.reshape(1, 128)
    b2 = gin_b2.reshape(1, 128)

    A = _tc0(emb_gcn, emb_gin, dego, gcn_W[0])
    out = None
    for i in range(3):
        M = _sc_pass(e2d_p, A, zrows)
        eps = gin_eps[i].reshape(1, 1)
        gcnb = gcn_b[i].reshape(1, 128)
        if i < 2:
            A = _tc_layer(M, A, dego, degi, gcnb, eps,
                          gin_W1, b1, gin_W2, b2, gcn_W[i + 1])
        else:
            out = _tc_final(M, A, degi, gcnb, eps, gin_W1, b1, gin_W2, b2,
                            gids3d, ffnn_W0, ffnn_b0.reshape(1, 128),
                            ffnn_W1, ffnn_b1.reshape(1, 128),
                            fc_W, fc_b.reshape(1, 1))
    return out.reshape(G, 1, 1)
